# baseline (speedup 1.0000x reference)
import jax
import jax.numpy as jnp
from jax.experimental import pallas as pl
from jax.experimental.pallas import tpu as pltpu

N = 1024

_TDIMS = (((0,), (0,)), ((), ()))


def _aggT(adjb, y):
    y_hi = y.astype(jnp.bfloat16)
    y_lo = (y - y_hi.astype(jnp.float32)).astype(jnp.bfloat16)
    z = jax.lax.dot_general(adjb, y_hi, _TDIMS,
                            preferred_element_type=jnp.float32)
    z += jax.lax.dot_general(adjb, y_lo, _TDIMS,
                             preferred_element_type=jnp.float32)
    return z + 2.0 * y


def _fused_kernel(adj_ref, emb_ref, w1_ref, b1_ref, w2_ref, b2_ref,
                  fc1w_ref, fc1b_ref, fc2w_ref, fc2b_ref, x_out_ref, a2_out_ref):
    adjb = adj_ref[...].astype(jnp.bfloat16)

    ones_col = jnp.ones((N, 1), jnp.bfloat16)
    deg = jax.lax.dot_general(adjb, ones_col, _TDIMS,
                              preferred_element_type=jnp.float32) + 2.0
    dis = jax.lax.rsqrt(deg)

    xt = jnp.dot(emb_ref[...], w1_ref[...], preferred_element_type=jnp.float32)
    x = jnp.maximum(dis * _aggT(adjb, dis * xt) + b1_ref[...], 0.0)

    xt2 = jnp.dot(x, w2_ref[...], preferred_element_type=jnp.float32)
    x2 = jnp.maximum(dis * _aggT(adjb, dis * xt2) + b2_ref[...], 0.0)
    x_out_ref[...] = x2

    h = jnp.maximum(jnp.dot(x2, fc1w_ref[...], preferred_element_type=jnp.float32)
                    + fc1b_ref[...], 0.0)
    a2_out_ref[...] = (jnp.dot(h, fc2w_ref[...], preferred_element_type=jnp.float32)
                       + fc2b_ref[...])


def kernel(adj, node_emb, W1, b1, W2, b2, fc1_W, fc1_b, fc2_W, fc2_b):
    x, a2 = pl.pallas_call(
        _fused_kernel,
        out_shape=(
            jax.ShapeDtypeStruct((N, 128), jnp.float32),
            jax.ShapeDtypeStruct((N, 1), jnp.float32),
        ),
    )(adj, node_emb, W1, b1.reshape(1, 512), W2, b2.reshape(1, 128),
      fc1_W, fc1_b.reshape(1, 256), fc2_W, fc2_b.reshape(1, 1))
    return (x, a2)


# speedup vs baseline: 1.1097x; 1.1097x over previous
import jax
import jax.numpy as jnp
from jax.experimental import pallas as pl
from jax.experimental.pallas import tpu as pltpu

N = 1024

_TDIMS = (((0,), (0,)), ((), ()))


def _aggT(adjb, y):
    z = jax.lax.dot_general(adjb, y, _TDIMS,
                            preferred_element_type=jnp.float32)
    return z + 2.0 * y


def _fused_kernel(adj_ref, emb_ref, w1_ref, b1_ref, w2_ref, b2_ref,
                  fc1w_ref, fc1b_ref, fc2w_ref, fc2b_ref, x_out_ref, a2_out_ref):
    adjb = adj_ref[...]

    ones_col = jnp.ones((N, 1), jnp.float32)
    deg = jax.lax.dot_general(adjb, ones_col, _TDIMS,
                              preferred_element_type=jnp.float32) + 2.0
    dis = jax.lax.rsqrt(deg)

    xt = jnp.dot(emb_ref[...], w1_ref[...], preferred_element_type=jnp.float32)
    x = jnp.maximum(dis * _aggT(adjb, dis * xt) + b1_ref[...], 0.0)

    xt2 = jnp.dot(x, w2_ref[...], preferred_element_type=jnp.float32)
    x2 = jnp.maximum(dis * _aggT(adjb, dis * xt2) + b2_ref[...], 0.0)
    x_out_ref[...] = x2

    h = jnp.maximum(jnp.dot(x2, fc1w_ref[...], preferred_element_type=jnp.float32)
                    + fc1b_ref[...], 0.0)
    a2_out_ref[...] = (jnp.dot(h, fc2w_ref[...], preferred_element_type=jnp.float32)
                       + fc2b_ref[...])


def kernel(adj, node_emb, W1, b1, W2, b2, fc1_W, fc1_b, fc2_W, fc2_b):
    x, a2 = pl.pallas_call(
        _fused_kernel,
        out_shape=(
            jax.ShapeDtypeStruct((N, 128), jnp.float32),
            jax.ShapeDtypeStruct((N, 1), jnp.float32),
        ),
    )(adj, node_emb, W1, b1.reshape(1, 512), W2, b2.reshape(1, 128),
      fc1_W, fc1_b.reshape(1, 256), fc2_W, fc2_b.reshape(1, 1))
    return (x, a2)
